# baseline (speedup 1.0000x reference)
import functools

import jax
import jax.numpy as jnp
from jax import lax
from jax.experimental import pallas as pl
from jax.experimental.pallas import tpu as pltpu
from jax.experimental.pallas import tpu_sc as plsc

_NSUB = 16
_NCORES = 2
_LANES = 16


def _mlp_head(embed, w1a, w1b, b1row):
    n, d = embed.shape
    h = w1a.shape[1]
    br = 400
    assert n % br == 0

    def body(e_ref, wa_ref, wb_ref, b1_ref, p_ref, q_ref):
        e = e_ref[...]
        p_ref[...] = (
            jnp.dot(e, wa_ref[...], preferred_element_type=jnp.float32)
            + b1_ref[...]
        )
        q_ref[...] = jnp.dot(e, wb_ref[...], preferred_element_type=jnp.float32)

    return pl.pallas_call(
        body,
        grid=(n // br,),
        in_specs=[
            pl.BlockSpec((br, d), lambda i: (i, 0)),
            pl.BlockSpec((d, h), lambda i: (0, 0)),
            pl.BlockSpec((d, h), lambda i: (0, 0)),
            pl.BlockSpec((1, h), lambda i: (0, 0)),
        ],
        out_specs=[
            pl.BlockSpec((br, h), lambda i: (i, 0)),
            pl.BlockSpec((br, h), lambda i: (i, 0)),
        ],
        out_shape=[
            jax.ShapeDtypeStruct((n, h), jnp.float32),
            jax.ShapeDtypeStruct((n, h), jnp.float32),
        ],
    )(embed, w1a, w1b, b1row)


def _gather_sum(p, q, col, row, e):
    n, h = p.shape
    nw = _NCORES * _NSUB
    et = e // nw
    ch = 800
    chunks = []
    off = 0
    while off < et:
        sz = min(ch, et - off)
        chunks.append((off, sz))
        off += sz
    mesh = plsc.VectorSubcoreMesh(core_axis_name="c", subcore_axis_name="s")

    @functools.partial(
        pl.kernel,
        out_type=jax.ShapeDtypeStruct((e, h), jnp.float32),
        mesh=mesh,
        scratch_types=[
            pltpu.VMEM((et,), jnp.int32),
            pltpu.VMEM((et,), jnp.int32),
            pltpu.VMEM((ch, h), jnp.float32),
            pltpu.VMEM((ch, h), jnp.float32),
        ],
        compiler_params=pltpu.CompilerParams(use_tc_tiling_on_sc=False),
    )
    def kern(p_hbm, q_hbm, col_hbm, row_hbm, g_hbm,
             colv, rowv, bufa, bufb):
        c = lax.axis_index("c")
        s = lax.axis_index("s")
        wid = c * _NSUB + s
        base = wid * et
        pltpu.sync_copy(col_hbm.at[pl.ds(base, et)], colv)
        pltpu.sync_copy(row_hbm.at[pl.ds(base, et)], rowv)
        nv = h // _LANES
        for off, sz in chunks:
            pltpu.sync_copy(p_hbm.at[colv.at[pl.ds(off, sz)]],
                            bufa.at[pl.ds(0, sz), :])
            pltpu.sync_copy(q_hbm.at[rowv.at[pl.ds(off, sz)]],
                            bufb.at[pl.ds(0, sz), :])

            def add_row(i, _, _nv=nv):
                for j in range(_nv):
                    sl = pl.ds(j * _LANES, _LANES)
                    bufa[i, sl] = bufa[i, sl] + bufb[i, sl]
                return 0

            lax.fori_loop(0, sz, add_row, 0)
            pltpu.sync_copy(bufa.at[pl.ds(0, sz), :],
                            g_hbm.at[pl.ds(base + off, sz), :])

    return kern(p, q, col, row)


def _edge_score(g, w2row, b2v, e):
    h = g.shape[1]
    br = 2048

    def body(g_ref, w2_ref, b2_ref, o_ref):
        a = jnp.maximum(g_ref[...], 0.0)
        v = jnp.sum(a * w2_ref[...], axis=1) + b2_ref[0, 0]
        o_ref[...] = jax.nn.sigmoid(v)

    return pl.pallas_call(
        body,
        grid=(pl.cdiv(e, br),),
        in_specs=[
            pl.BlockSpec((br, h), lambda i: (i, 0)),
            pl.BlockSpec((1, h), lambda i: (0, 0)),
            pl.BlockSpec((1, 1), lambda i: (0, 0)),
        ],
        out_specs=pl.BlockSpec((br,), lambda i: (i,)),
        out_shape=jax.ShapeDtypeStruct((e,), jnp.float32),
    )(g, w2row, b2v)


_HASH_MULS = (-1640531527, -2048144789, -1028477387, 668265263,
              374761393, -1700995253, 1181783497)
_TBITS = 17
_TSIZE = 1 << _TBITS


def _dedup_sum(col, row, values, e, n):
    et = e // _NSUB
    npad = ((et + 127) // 128) * 128
    nvr = npad // _LANES
    nreal = et // _LANES
    unroll = 4
    assert nvr % unroll == 0
    mesh = plsc.VectorSubcoreMesh(core_axis_name="c", subcore_axis_name="s")

    @functools.partial(
        pl.kernel,
        out_type=jax.ShapeDtypeStruct((e + 2 * _NSUB,), jnp.float32),
        mesh=mesh,
        scratch_types=[
            pltpu.VMEM((npad,), jnp.int32),
            pltpu.VMEM((npad,), jnp.int32),
            pltpu.VMEM((npad,), jnp.int32),
            pltpu.VMEM((npad,), jnp.int32),
            pltpu.VMEM((npad,), jnp.int32),
            pltpu.VMEM((npad,), jnp.float32),
            pltpu.VMEM((npad,), jnp.int32),
            pltpu.VMEM((npad,), jnp.float32),
            pltpu.VMEM((npad,), jnp.float32),
            pltpu.VMEM((npad,), jnp.float32),
            pltpu.VMEM((npad,), jnp.float32),
            pltpu.VMEM((_LANES,), jnp.int32),
            pltpu.VMEM((_NSUB * _LANES,), jnp.int32),
            pltpu.VMEM((_LANES,), jnp.int32),
            pltpu.VMEM_SHARED((_TSIZE + _NSUB,), jnp.int32),
            pltpu.VMEM_SHARED((_TSIZE + _NSUB,), jnp.float32),
            pltpu.VMEM_SHARED((_NSUB * _LANES,), jnp.int32),
        ],
        compiler_params=pltpu.CompilerParams(use_tc_tiling_on_sc=False),
    )
    def kern(col_hbm, row_hbm, v_hbm, o_hbm, colb, rowb, oidxb, keyb, hb,
             valb, gib, gvb, addb, outb, zb, cntv, cntall, totv, ktab, vtab,
             cntsh):
        c = lax.axis_index("c")
        s = lax.axis_index("s")

        def _():
            base = s * et
            wid = c * _NSUB + s
            dumpo = jnp.zeros((_LANES,), jnp.int32) + (e + wid)
            iotav = lax.iota(jnp.int32, _LANES)
            basev = jnp.zeros((_LANES,), jnp.int32) + base
            cv = c
            pltpu.sync_copy(col_hbm.at[pl.ds(base, et)],
                            colb.at[pl.ds(0, et)])
            pltpu.sync_copy(row_hbm.at[pl.ds(base, et)],
                            rowb.at[pl.ds(0, et)])
            pltpu.sync_copy(v_hbm.at[pl.ds(base, et)],
                            valb.at[pl.ds(0, et)])
            dump = _TSIZE + s
            dumpv = jnp.zeros((_LANES,), jnp.int32) + dump
            zv = jnp.zeros((_LANES,), jnp.float32)
            neg1 = jnp.full((_LANES,), -1, jnp.int32)
            mul0 = jnp.int32(_HASH_MULS[0])
            shift = jnp.int32(32 - _TBITS)

            ownmul = jnp.int32(658356373)

            def init_one(i, sl):
                k = colb[sl] * n + rowb[sl]
                own = lax.shift_right_logical(k * ownmul,
                                              jnp.int32(31)) == cv
                k = jnp.where(own, k, neg1)
                keyb[sl] = k
                hh = lax.shift_right_logical(k * mul0, shift)
                hb[sl] = jnp.where(own, hh, dumpv)
                oidxb[sl] = jnp.where(own, basev + iotav + i * _LANES,
                                      dumpo)
                outb[sl] = zv
                zb[sl] = zv

            def init_body(i, _):
                for u in range(unroll):
                    j = i * unroll + u
                    init_one(j, pl.ds(j * _LANES, _LANES))
                return 0

            lax.fori_loop(0, nreal // unroll, init_body, 0)
            for i in range((nreal // unroll) * unroll, nreal):
                init_one(i, pl.ds(i * _LANES, _LANES))
            for i in range(nreal, nvr):
                sl = pl.ds(i * _LANES, _LANES)
                keyb[sl] = neg1
                hb[sl] = dumpv
                oidxb[sl] = dumpo
                valb[sl] = zv
                outb[sl] = zv
                zb[sl] = zv

            for lvl in range(len(_HASH_MULS)):
                last = lvl == len(_HASH_MULS) - 1
                muln = jnp.int32(_HASH_MULS[min(lvl + 1, len(_HASH_MULS) - 1)])

                zchunk = _TSIZE // _NSUB

                def level_body(_muln=muln, _last=last):
                    pltpu.sync_copy(keyb, ktab.at[hb])
                    pltpu.sync_copy(zb.at[pl.ds(0, zchunk)],
                                    vtab.at[pl.ds(s * zchunk, zchunk)])
                    plsc.subcore_barrier()
                    pltpu.sync_copy(ktab.at[hb], gib)

                    def addsrc_body(i, _):
                        for u in range(unroll):
                            sl = pl.ds((i * unroll + u) * _LANES, _LANES)
                            k = keyb[sl]
                            w = (gib[sl] == k) & (k >= 0)
                            addb[sl] = jnp.where(w, valb[sl], zv)
                        return 0

                    lax.fori_loop(0, nvr // unroll, addsrc_body, 0)
                    pltpu.sync_copy(addb, vtab.at[hb], add=True)
                    plsc.subcore_barrier()
                    pltpu.sync_copy(vtab.at[hb], gvb)

                    one16 = jnp.full((_LANES,), 1, jnp.int32)
                    zero16 = jnp.zeros((_LANES,), jnp.int32)

                    def retire_body(i, cnt):
                        for u in range(unroll):
                            sl = pl.ds((i * unroll + u) * _LANES, _LANES)
                            k = keyb[sl]
                            w = (gib[sl] == k) & (k >= 0)
                            outb[sl] = jnp.where(w, gvb[sl], outb[sl])
                            k2 = jnp.where(w, neg1, k)
                            keyb[sl] = k2
                            if not _last:
                                hh = lax.shift_right_logical(k2 * _muln,
                                                             shift)
                                hb[sl] = jnp.where(k2 >= 0, hh, dumpv)
                                cnt = cnt + jnp.where(k2 >= 0, one16, zero16)
                        return cnt

                    cnt = lax.fori_loop(0, nvr // unroll, retire_body, zero16)
                    if not _last:
                        cntv[pl.ds(0, _LANES)] = cnt
                    plsc.subcore_barrier()

                if lvl == 0:
                    level_body()
                else:
                    pltpu.sync_copy(cntv, cntsh.at[pl.ds(s * _LANES, _LANES)])
                    plsc.subcore_barrier()
                    pltpu.sync_copy(cntsh, cntall)
                    acc = jnp.zeros((_LANES,), jnp.int32)
                    for i in range(_NSUB):
                        acc = acc + cntall[pl.ds(i * _LANES, _LANES)]
                    totv[pl.ds(0, _LANES)] = acc
                    av = totv[pl.ds(0, _LANES)]
                    t = av[0]
                    for i in range(1, _LANES):
                        t = t + av[i]
                    pl.when(t > 0)(level_body)

            pltpu.sync_copy(outb, o_hbm.at[oidxb])

        _()

    return kern(col, row, values)


def kernel(embed, edge_index, W1, b1, W2, b2):
    n, d = embed.shape
    e = edge_index.shape[1]
    h = W1.shape[1]
    w1a = W1[:d]
    w1b = W1[d:]
    b1row = b1.reshape(1, h)
    w2row = W2.reshape(1, h)
    b2v = b2.reshape(1, 1)
    col = edge_index[0]
    row = edge_index[1]
    p, q = _mlp_head(embed, w1a, w1b, b1row)
    g = _gather_sum(p, q, col, row, e)
    values = _edge_score(g, w2row, b2v, e)
    return _dedup_sum(col, row, values, e, n)[:e]


# speedup vs baseline: 37.8005x; 37.8005x over previous
import functools

import jax
import jax.numpy as jnp
from jax import lax
from jax.experimental import pallas as pl
from jax.experimental.pallas import tpu as pltpu
from jax.experimental.pallas import tpu_sc as plsc

_NSUB = 16
_NCORES = 2
_LANES = 16


def _mlp_head(embed, w1a, w1b, b1row):
    n, d = embed.shape
    h = w1a.shape[1]
    br = 400
    assert n % br == 0

    def body(e_ref, wa_ref, wb_ref, b1_ref, p_ref, q_ref):
        e = e_ref[...]
        p_ref[...] = (
            jnp.dot(e, wa_ref[...], preferred_element_type=jnp.float32)
            + b1_ref[...]
        )
        q_ref[...] = jnp.dot(e, wb_ref[...], preferred_element_type=jnp.float32)

    return pl.pallas_call(
        body,
        grid=(n // br,),
        in_specs=[
            pl.BlockSpec((br, d), lambda i: (i, 0)),
            pl.BlockSpec((d, h), lambda i: (0, 0)),
            pl.BlockSpec((d, h), lambda i: (0, 0)),
            pl.BlockSpec((1, h), lambda i: (0, 0)),
        ],
        out_specs=[
            pl.BlockSpec((br, h), lambda i: (i, 0)),
            pl.BlockSpec((br, h), lambda i: (i, 0)),
        ],
        out_shape=[
            jax.ShapeDtypeStruct((n, h), jnp.float32),
            jax.ShapeDtypeStruct((n, h), jnp.float32),
        ],
    )(embed, w1a, w1b, b1row)


def _gather_sum(p, q, col, row, e):
    n, h = p.shape
    nw = _NCORES * _NSUB
    et = e // nw
    ch = 800
    chunks = []
    off = 0
    while off < et:
        sz = min(ch, et - off)
        chunks.append((off, sz))
        off += sz
    mesh = plsc.VectorSubcoreMesh(core_axis_name="c", subcore_axis_name="s")

    @functools.partial(
        pl.kernel,
        out_type=jax.ShapeDtypeStruct((e, h), jnp.float32),
        mesh=mesh,
        scratch_types=[
            pltpu.VMEM((et,), jnp.int32),
            pltpu.VMEM((et,), jnp.int32),
            pltpu.VMEM((ch, h), jnp.float32),
            pltpu.VMEM((ch, h), jnp.float32),
        ],
        compiler_params=pltpu.CompilerParams(use_tc_tiling_on_sc=False),
    )
    def kern(p_hbm, q_hbm, col_hbm, row_hbm, g_hbm,
             colv, rowv, bufa, bufb):
        c = lax.axis_index("c")
        s = lax.axis_index("s")
        wid = c * _NSUB + s
        base = wid * et
        pltpu.sync_copy(col_hbm.at[pl.ds(base, et)], colv)
        pltpu.sync_copy(row_hbm.at[pl.ds(base, et)], rowv)
        nv = h // _LANES
        for off, sz in chunks:
            pltpu.sync_copy(p_hbm.at[colv.at[pl.ds(off, sz)]],
                            bufa.at[pl.ds(0, sz), :])
            pltpu.sync_copy(q_hbm.at[rowv.at[pl.ds(off, sz)]],
                            bufb.at[pl.ds(0, sz), :])

            def add_row(i, _, _nv=nv):
                for j in range(_nv):
                    sl = pl.ds(j * _LANES, _LANES)
                    bufa[i, sl] = bufa[i, sl] + bufb[i, sl]
                return 0

            lax.fori_loop(0, sz, add_row, 0)
            pltpu.sync_copy(bufa.at[pl.ds(0, sz), :],
                            g_hbm.at[pl.ds(base + off, sz), :])

    return kern(p, q, col, row)


def _edge_score(g, w2row, b2v, e):
    h = g.shape[1]
    br = 2048

    def body(g_ref, w2_ref, b2_ref, o_ref):
        a = jnp.maximum(g_ref[...], 0.0)
        v = jnp.sum(a * w2_ref[...], axis=1) + b2_ref[0, 0]
        o_ref[...] = jax.nn.sigmoid(v)

    return pl.pallas_call(
        body,
        grid=(pl.cdiv(e, br),),
        in_specs=[
            pl.BlockSpec((br, h), lambda i: (i, 0)),
            pl.BlockSpec((1, h), lambda i: (0, 0)),
            pl.BlockSpec((1, 1), lambda i: (0, 0)),
        ],
        out_specs=pl.BlockSpec((br,), lambda i: (i,)),
        out_shape=jax.ShapeDtypeStruct((e,), jnp.float32),
    )(g, w2row, b2v)


_HASH_MULS = (-1640531527, -2048144789, -1028477387, 668265263,
              374761393, -1700995253, 1181783497)
_TBITS = 17
_TSIZE = 1 << _TBITS


def _dedup_sum(col, row, values, e, n):
    et = e // _NSUB
    npad = ((et + 127) // 128) * 128
    nvr = npad // _LANES
    nreal = et // _LANES
    unroll = 4
    assert nvr % unroll == 0
    mesh = plsc.VectorSubcoreMesh(core_axis_name="c", subcore_axis_name="s")

    @functools.partial(
        pl.kernel,
        out_type=jax.ShapeDtypeStruct((2, e), jnp.float32),
        mesh=mesh,
        scratch_types=[
            pltpu.VMEM((npad,), jnp.int32),
            pltpu.VMEM((npad,), jnp.int32),
            pltpu.VMEM((npad,), jnp.int32),
            pltpu.VMEM((npad,), jnp.int32),
            pltpu.VMEM((npad,), jnp.float32),
            pltpu.VMEM((npad,), jnp.int32),
            pltpu.VMEM((npad,), jnp.float32),
            pltpu.VMEM((npad,), jnp.float32),
            pltpu.VMEM((npad,), jnp.float32),
            pltpu.VMEM((npad,), jnp.float32),
            pltpu.VMEM((_LANES,), jnp.int32),
            pltpu.VMEM((_NSUB * _LANES,), jnp.int32),
            pltpu.VMEM((_LANES,), jnp.int32),
            pltpu.VMEM_SHARED((_TSIZE + _NSUB,), jnp.int32),
            pltpu.VMEM_SHARED((_TSIZE + _NSUB,), jnp.float32),
            pltpu.VMEM_SHARED((_NSUB * _LANES,), jnp.int32),
        ],
        compiler_params=pltpu.CompilerParams(use_tc_tiling_on_sc=False),
    )
    def kern(col_hbm, row_hbm, v_hbm, o_hbm, colb, rowb, keyb, hb,
             valb, gib, gvb, addb, outb, zb, cntv, cntall, totv, ktab, vtab,
             cntsh):
        c = lax.axis_index("c")
        s = lax.axis_index("s")

        def _():
            base = s * et
            cv = c
            pltpu.sync_copy(col_hbm.at[pl.ds(base, et)],
                            colb.at[pl.ds(0, et)])
            pltpu.sync_copy(row_hbm.at[pl.ds(base, et)],
                            rowb.at[pl.ds(0, et)])
            pltpu.sync_copy(v_hbm.at[pl.ds(base, et)],
                            valb.at[pl.ds(0, et)])
            dump = _TSIZE + s
            dumpv = jnp.zeros((_LANES,), jnp.int32) + dump
            zv = jnp.zeros((_LANES,), jnp.float32)
            neg1 = jnp.full((_LANES,), -1, jnp.int32)
            mul0 = jnp.int32(_HASH_MULS[0])
            shift = jnp.int32(32 - _TBITS)

            ownmul = jnp.int32(658356373)

            def init_one(i, sl):
                k = colb[sl] * n + rowb[sl]
                own = lax.shift_right_logical(k * ownmul,
                                              jnp.int32(31)) == cv
                k = jnp.where(own, k, neg1)
                keyb[sl] = k
                hh = lax.shift_right_logical(k * mul0, shift)
                hb[sl] = jnp.where(own, hh, dumpv)
                outb[sl] = zv
                zb[sl] = zv

            def init_body(i, _):
                for u in range(unroll):
                    j = i * unroll + u
                    init_one(j, pl.ds(j * _LANES, _LANES))
                return 0

            lax.fori_loop(0, nreal // unroll, init_body, 0)
            for i in range((nreal // unroll) * unroll, nreal):
                init_one(i, pl.ds(i * _LANES, _LANES))
            for i in range(nreal, nvr):
                sl = pl.ds(i * _LANES, _LANES)
                keyb[sl] = neg1
                hb[sl] = dumpv
                valb[sl] = zv
                outb[sl] = zv
                zb[sl] = zv

            for lvl in range(len(_HASH_MULS)):
                last = lvl == len(_HASH_MULS) - 1
                muln = jnp.int32(_HASH_MULS[min(lvl + 1, len(_HASH_MULS) - 1)])

                zchunk = _TSIZE // _NSUB

                def level_body(_muln=muln, _last=last):
                    pltpu.sync_copy(keyb, ktab.at[hb])
                    pltpu.sync_copy(zb.at[pl.ds(0, zchunk)],
                                    vtab.at[pl.ds(s * zchunk, zchunk)])
                    plsc.subcore_barrier()
                    pltpu.sync_copy(ktab.at[hb], gib)

                    def addsrc_body(i, _):
                        for u in range(unroll):
                            sl = pl.ds((i * unroll + u) * _LANES, _LANES)
                            k = keyb[sl]
                            w = (gib[sl] == k) & (k >= 0)
                            addb[sl] = jnp.where(w, valb[sl], zv)
                        return 0

                    lax.fori_loop(0, nvr // unroll, addsrc_body, 0)
                    pltpu.sync_copy(addb, vtab.at[hb], add=True)
                    plsc.subcore_barrier()
                    pltpu.sync_copy(vtab.at[hb], gvb)

                    one16 = jnp.full((_LANES,), 1, jnp.int32)
                    zero16 = jnp.zeros((_LANES,), jnp.int32)

                    def retire_body(i, cnt):
                        for u in range(unroll):
                            sl = pl.ds((i * unroll + u) * _LANES, _LANES)
                            k = keyb[sl]
                            w = (gib[sl] == k) & (k >= 0)
                            outb[sl] = jnp.where(w, gvb[sl], outb[sl])
                            k2 = jnp.where(w, neg1, k)
                            keyb[sl] = k2
                            if not _last:
                                hh = lax.shift_right_logical(k2 * _muln,
                                                             shift)
                                hb[sl] = jnp.where(k2 >= 0, hh, dumpv)
                                cnt = cnt + jnp.where(k2 >= 0, one16, zero16)
                        return cnt

                    cnt = lax.fori_loop(0, nvr // unroll, retire_body, zero16)
                    if not _last:
                        cntv[pl.ds(0, _LANES)] = cnt
                    plsc.subcore_barrier()

                if lvl == 0:
                    level_body()
                else:
                    pltpu.sync_copy(cntv, cntsh.at[pl.ds(s * _LANES, _LANES)])
                    plsc.subcore_barrier()
                    pltpu.sync_copy(cntsh, cntall)
                    acc = jnp.zeros((_LANES,), jnp.int32)
                    for i in range(_NSUB):
                        acc = acc + cntall[pl.ds(i * _LANES, _LANES)]
                    totv[pl.ds(0, _LANES)] = acc
                    av = totv[pl.ds(0, _LANES)]
                    t = av[0]
                    for i in range(1, _LANES):
                        t = t + av[i]
                    pl.when(t > 0)(level_body)

            pltpu.sync_copy(outb.at[pl.ds(0, et)],
                            o_hbm.at[c, pl.ds(base, et)])

        _()

    return kern(col, row, values)


def _pair_sum(o2, e):
    br = 2048

    def body(o2_ref, o_ref):
        o_ref[...] = o2_ref[0, :] + o2_ref[1, :]

    return pl.pallas_call(
        body,
        grid=(pl.cdiv(e, br),),
        in_specs=[pl.BlockSpec((2, br), lambda i: (0, i))],
        out_specs=pl.BlockSpec((br,), lambda i: (i,)),
        out_shape=jax.ShapeDtypeStruct((e,), jnp.float32),
    )(o2)


def kernel(embed, edge_index, W1, b1, W2, b2):
    n, d = embed.shape
    e = edge_index.shape[1]
    h = W1.shape[1]
    w1a = W1[:d]
    w1b = W1[d:]
    b1row = b1.reshape(1, h)
    w2row = W2.reshape(1, h)
    b2v = b2.reshape(1, 1)
    col = edge_index[0]
    row = edge_index[1]
    p, q = _mlp_head(embed, w1a, w1b, b1row)
    g = _gather_sum(p, q, col, row, e)
    values = _edge_score(g, w2row, b2v, e)
    return _pair_sum(_dedup_sum(col, row, values, e, n), e)


# speedup vs baseline: 38.6873x; 1.0235x over previous
import functools

import jax
import jax.numpy as jnp
from jax import lax
from jax.experimental import pallas as pl
from jax.experimental.pallas import tpu as pltpu
from jax.experimental.pallas import tpu_sc as plsc

_NSUB = 16
_NCORES = 2
_LANES = 16


def _mlp_head(embed, w1a, w1b, b1row):
    n, d = embed.shape
    h = w1a.shape[1]
    br = 400
    assert n % br == 0

    def body(e_ref, wa_ref, wb_ref, b1_ref, p_ref, q_ref):
        e = e_ref[...]
        p_ref[...] = (
            jnp.dot(e, wa_ref[...], preferred_element_type=jnp.float32)
            + b1_ref[...]
        )
        q_ref[...] = jnp.dot(e, wb_ref[...], preferred_element_type=jnp.float32)

    return pl.pallas_call(
        body,
        grid=(n // br,),
        in_specs=[
            pl.BlockSpec((br, d), lambda i: (i, 0)),
            pl.BlockSpec((d, h), lambda i: (0, 0)),
            pl.BlockSpec((d, h), lambda i: (0, 0)),
            pl.BlockSpec((1, h), lambda i: (0, 0)),
        ],
        out_specs=[
            pl.BlockSpec((br, h), lambda i: (i, 0)),
            pl.BlockSpec((br, h), lambda i: (i, 0)),
        ],
        out_shape=[
            jax.ShapeDtypeStruct((n, h), jnp.float32),
            jax.ShapeDtypeStruct((n, h), jnp.float32),
        ],
    )(embed, w1a, w1b, b1row)


def _gather_sum(p, q, col, row, e):
    n, h = p.shape
    nw = _NCORES * _NSUB
    et = e // nw
    ch = 800
    chunks = []
    off = 0
    while off < et:
        sz = min(ch, et - off)
        chunks.append((off, sz))
        off += sz
    mesh = plsc.VectorSubcoreMesh(core_axis_name="c", subcore_axis_name="s")

    @functools.partial(
        pl.kernel,
        out_type=jax.ShapeDtypeStruct((e, h), jnp.float32),
        mesh=mesh,
        scratch_types=[
            pltpu.VMEM((et,), jnp.int32),
            pltpu.VMEM((et,), jnp.int32),
            pltpu.VMEM((ch, h), jnp.float32),
            pltpu.VMEM((ch, h), jnp.float32),
        ],
        compiler_params=pltpu.CompilerParams(use_tc_tiling_on_sc=False),
    )
    def kern(p_hbm, q_hbm, col_hbm, row_hbm, g_hbm,
             colv, rowv, bufa, bufb):
        c = lax.axis_index("c")
        s = lax.axis_index("s")
        wid = c * _NSUB + s
        base = wid * et
        pltpu.sync_copy(col_hbm.at[pl.ds(base, et)], colv)
        pltpu.sync_copy(row_hbm.at[pl.ds(base, et)], rowv)
        nv = h // _LANES
        for off, sz in chunks:
            pltpu.sync_copy(p_hbm.at[colv.at[pl.ds(off, sz)]],
                            bufa.at[pl.ds(0, sz), :])
            pltpu.sync_copy(q_hbm.at[rowv.at[pl.ds(off, sz)]],
                            bufb.at[pl.ds(0, sz), :])

            def add_rows(i, _, _nv=nv):
                for r in range(4):
                    for j in range(_nv):
                        sl = pl.ds(j * _LANES, _LANES)
                        bufa[i * 4 + r, sl] = (bufa[i * 4 + r, sl]
                                               + bufb[i * 4 + r, sl])
                return 0

            assert sz % 4 == 0
            lax.fori_loop(0, sz // 4, add_rows, 0)
            pltpu.sync_copy(bufa.at[pl.ds(0, sz), :],
                            g_hbm.at[pl.ds(base + off, sz), :])

    return kern(p, q, col, row)


def _edge_score(g, w2row, b2v, e):
    h = g.shape[1]
    br = 2048

    def body(g_ref, w2_ref, b2_ref, o_ref):
        a = jnp.maximum(g_ref[...], 0.0)
        v = jnp.sum(a * w2_ref[...], axis=1) + b2_ref[0, 0]
        o_ref[...] = jax.nn.sigmoid(v)

    return pl.pallas_call(
        body,
        grid=(pl.cdiv(e, br),),
        in_specs=[
            pl.BlockSpec((br, h), lambda i: (i, 0)),
            pl.BlockSpec((1, h), lambda i: (0, 0)),
            pl.BlockSpec((1, 1), lambda i: (0, 0)),
        ],
        out_specs=pl.BlockSpec((br,), lambda i: (i,)),
        out_shape=jax.ShapeDtypeStruct((e,), jnp.float32),
    )(g, w2row, b2v)


_HASH_MULS = (-1640531527, -2048144789, -1028477387, 668265263,
              374761393, -1700995253, 1181783497)
_TBITS = 17
_TSIZE = 1 << _TBITS


def _dedup_sum(col, row, values, e, n):
    et = e // _NSUB
    npad = ((et + 127) // 128) * 128
    nvr = npad // _LANES
    nreal = et // _LANES
    unroll = 4
    assert nvr % unroll == 0
    mesh = plsc.VectorSubcoreMesh(core_axis_name="c", subcore_axis_name="s")

    @functools.partial(
        pl.kernel,
        out_type=jax.ShapeDtypeStruct((e,), jnp.float32),
        mesh=mesh,
        scratch_types=[
            pltpu.VMEM((npad,), jnp.int32),
            pltpu.VMEM((npad,), jnp.int32),
            pltpu.VMEM((npad,), jnp.int32),
            pltpu.VMEM((npad,), jnp.int32),
            pltpu.VMEM((npad,), jnp.float32),
            pltpu.VMEM((npad,), jnp.int32),
            pltpu.VMEM((npad,), jnp.float32),
            pltpu.VMEM((npad,), jnp.float32),
            pltpu.VMEM((npad,), jnp.float32),
            pltpu.VMEM((npad,), jnp.float32),
            pltpu.VMEM((_LANES,), jnp.int32),
            pltpu.VMEM((_NSUB * _LANES,), jnp.int32),
            pltpu.VMEM((_LANES,), jnp.int32),
            pltpu.VMEM_SHARED((_TSIZE + _NSUB,), jnp.int32),
            pltpu.VMEM_SHARED((_TSIZE + _NSUB,), jnp.float32),
            pltpu.VMEM_SHARED((_NSUB * _LANES,), jnp.int32),
        ],
        compiler_params=pltpu.CompilerParams(use_tc_tiling_on_sc=False),
    )
    def kern(col_hbm, row_hbm, v_hbm, o_hbm, colb, rowb, keyb, hb,
             valb, gib, gvb, addb, outb, zb, cntv, cntall, totv, ktab, vtab,
             cntsh):
        c = lax.axis_index("c")
        s = lax.axis_index("s")

        def _():
            base = s * et
            pltpu.sync_copy(col_hbm.at[pl.ds(base, et)],
                            colb.at[pl.ds(0, et)])
            pltpu.sync_copy(row_hbm.at[pl.ds(base, et)],
                            rowb.at[pl.ds(0, et)])
            pltpu.sync_copy(v_hbm.at[pl.ds(base, et)],
                            valb.at[pl.ds(0, et)])
            dump = _TSIZE + s
            dumpv = jnp.zeros((_LANES,), jnp.int32) + dump
            zv = jnp.zeros((_LANES,), jnp.float32)
            neg1 = jnp.full((_LANES,), -1, jnp.int32)
            mul0 = jnp.int32(_HASH_MULS[0])
            shift = jnp.int32(32 - _TBITS)

            def init_one(i, sl):
                k = colb[sl] * n + rowb[sl]
                keyb[sl] = k
                hb[sl] = lax.shift_right_logical(k * mul0, shift)
                outb[sl] = zv
                zb[sl] = zv

            def init_body(i, _):
                for u in range(unroll):
                    j = i * unroll + u
                    init_one(j, pl.ds(j * _LANES, _LANES))
                return 0

            lax.fori_loop(0, nreal // unroll, init_body, 0)
            for i in range((nreal // unroll) * unroll, nreal):
                init_one(i, pl.ds(i * _LANES, _LANES))
            for i in range(nreal, nvr):
                sl = pl.ds(i * _LANES, _LANES)
                keyb[sl] = neg1
                hb[sl] = dumpv
                valb[sl] = zv
                outb[sl] = zv
                zb[sl] = zv

            for lvl in range(len(_HASH_MULS)):
                last = lvl == len(_HASH_MULS) - 1
                muln = jnp.int32(_HASH_MULS[min(lvl + 1, len(_HASH_MULS) - 1)])

                zchunk = _TSIZE // _NSUB

                def level_body(_muln=muln, _last=last):
                    pltpu.sync_copy(keyb, ktab.at[hb])
                    pltpu.sync_copy(zb.at[pl.ds(0, zchunk)],
                                    vtab.at[pl.ds(s * zchunk, zchunk)])
                    plsc.subcore_barrier()
                    pltpu.sync_copy(ktab.at[hb], gib)

                    def addsrc_body(i, _):
                        for u in range(unroll):
                            sl = pl.ds((i * unroll + u) * _LANES, _LANES)
                            k = keyb[sl]
                            w = (gib[sl] == k) & (k >= 0)
                            addb[sl] = jnp.where(w, valb[sl], zv)
                        return 0

                    lax.fori_loop(0, nvr // unroll, addsrc_body, 0)
                    pltpu.sync_copy(addb, vtab.at[hb], add=True)
                    plsc.subcore_barrier()
                    pltpu.sync_copy(vtab.at[hb], gvb)

                    one16 = jnp.full((_LANES,), 1, jnp.int32)
                    zero16 = jnp.zeros((_LANES,), jnp.int32)

                    def retire_body(i, cnt):
                        for u in range(unroll):
                            sl = pl.ds((i * unroll + u) * _LANES, _LANES)
                            k = keyb[sl]
                            w = (gib[sl] == k) & (k >= 0)
                            outb[sl] = jnp.where(w, gvb[sl], outb[sl])
                            k2 = jnp.where(w, neg1, k)
                            keyb[sl] = k2
                            if not _last:
                                hh = lax.shift_right_logical(k2 * _muln,
                                                             shift)
                                hb[sl] = jnp.where(k2 >= 0, hh, dumpv)
                                cnt = cnt + jnp.where(k2 >= 0, one16, zero16)
                        return cnt

                    cnt = lax.fori_loop(0, nvr // unroll, retire_body, zero16)
                    if not _last:
                        cntv[pl.ds(0, _LANES)] = cnt
                    plsc.subcore_barrier()

                if lvl == 0:
                    level_body()
                else:
                    pltpu.sync_copy(cntv, cntsh.at[pl.ds(s * _LANES, _LANES)])
                    plsc.subcore_barrier()
                    pltpu.sync_copy(cntsh, cntall)
                    acc = jnp.zeros((_LANES,), jnp.int32)
                    for i in range(_NSUB):
                        acc = acc + cntall[pl.ds(i * _LANES, _LANES)]
                    totv[pl.ds(0, _LANES)] = acc
                    av = totv[pl.ds(0, _LANES)]
                    t = av[0]
                    for i in range(1, _LANES):
                        t = t + av[i]
                    pl.when(t > 0)(level_body)

            pltpu.sync_copy(outb.at[pl.ds(0, et)],
                            o_hbm.at[pl.ds(base, et)])

        pl.when(c == 0)(_)

    return kern(col, row, values)


def kernel(embed, edge_index, W1, b1, W2, b2):
    n, d = embed.shape
    e = edge_index.shape[1]
    h = W1.shape[1]
    w1a = W1[:d]
    w1b = W1[d:]
    b1row = b1.reshape(1, h)
    w2row = W2.reshape(1, h)
    b2v = b2.reshape(1, 1)
    col = edge_index[0]
    row = edge_index[1]
    p, q = _mlp_head(embed, w1a, w1b, b1row)
    g = _gather_sum(p, q, col, row, e)
    values = _edge_score(g, w2row, b2v, e)
    return _dedup_sum(col, row, values, e, n)
